# baseline (speedup 1.0000x reference)
import functools

import jax
import jax.numpy as jnp
from jax import lax
from jax.experimental import pallas as pl
from jax.experimental.pallas import tpu as pltpu
from jax.experimental.pallas import tpu_sc as plsc

N_ROWS = 50000
N_COLS = 50000
N_NODES = N_ROWS + N_COLS
HALF = 50000
D = 32
N_EDGES = 1600000

NCORES = 2
NSUB = 16
ECHUNK = N_EDGES // NSUB
BLK = 2048
NFULL = ECHUNK // BLK
TAIL = ECHUNK - NFULL * BLK
RING = 4096
FLUSH = 1024
SUPER = 2048
CAP = ((ECHUNK + SUPER - 1) // SUPER + 1) * SUPER
NSLOT = CAP // 256
CH = 128
SLOT_CH = 2
RPT = 3128
RPT_LAST = HALF - (NSUB - 1) * RPT
ACC_ROWS = HALF + 8
DUMMY = HALF
NTILES = NCORES * NSUB


def _partition_body(src_h, dst_h, w_h, esrc_h, edst_h, ew_h, nsup_h,
                    sbuf, dbuf, wbuf, ring_s, ring_d, ring_w, cntbuf):
    c = lax.axis_index("c")
    s = lax.axis_index("s")
    lo = c * HALF
    lane = lax.iota(jnp.int32, 16)
    lbase = (c * NSUB + s) * CAP

    def flush(flushed):
        start = pl.multiple_of(flushed & (RING - 1), 8)
        pltpu.sync_copy(ring_s.at[pl.ds(start, FLUSH)],
                        esrc_h.at[pl.ds(pl.multiple_of(lbase + flushed, 8), FLUSH)])
        pltpu.sync_copy(ring_d.at[pl.ds(start, FLUSH)],
                        edst_h.at[pl.ds(pl.multiple_of(lbase + flushed, 8), FLUSH)])
        pltpu.sync_copy(ring_w.at[pl.ds(start, FLUSH)],
                        ew_h.at[pl.ds(pl.multiple_of(lbase + flushed, 8), FLUSH)])

    def vbody(i, carry):
        cnt, flushed = carry
        off = i * 16
        sv = sbuf[pl.ds(off, 16)]
        dv = dbuf[pl.ds(off, 16)]
        wv = wbuf[pl.ds(off, 16)]
        m = (dv >= lo) & (dv < lo + HALF)
        pc = plsc.cumsum(m.astype(jnp.int32))
        tot = jnp.max(pc)
        pos = (cnt + pc - 1) & (RING - 1)
        plsc.store_scatter(ring_s, [pos], sv, mask=m)
        plsc.store_scatter(ring_d, [pos], dv - lo, mask=m)
        plsc.store_scatter(ring_w, [pos], wv, mask=m)
        cnt = cnt + tot
        cond = (cnt - flushed) >= FLUSH
        pl.when(cond)(lambda: flush(flushed))
        flushed = jnp.where(cond, flushed + FLUSH, flushed)
        return cnt, flushed

    def stage(base, n):
        boff = s * ECHUNK + base
        pltpu.sync_copy(src_h.at[pl.ds(pl.multiple_of(boff, 8), n)], sbuf.at[pl.ds(0, n)])
        pltpu.sync_copy(dst_h.at[pl.ds(pl.multiple_of(boff, 8), n)], dbuf.at[pl.ds(0, n)])
        pltpu.sync_copy(w_h.at[pl.ds(pl.multiple_of(boff, 8), n)], wbuf.at[pl.ds(0, n)])

    def blk_body(b, carry):
        stage(b * BLK, BLK)
        return lax.fori_loop(0, BLK // 16, vbody, carry)

    cnt, flushed = lax.fori_loop(0, NFULL, blk_body,
                                 (jnp.int32(0), jnp.int32(0)))
    stage(NFULL * BLK, TAIL)
    cnt, flushed = lax.fori_loop(0, TAIL // 16, vbody, (cnt, flushed))

    npad = (SUPER - (cnt & (SUPER - 1))) & (SUPER - 1)
    zi = jnp.zeros((16,), jnp.int32)
    zf = jnp.zeros((16,), jnp.float32)
    dumv = jnp.full((16,), DUMMY, jnp.int32)

    def pad_body(i, _):
        idx = i * 16 + lane
        m = idx < npad
        pos = (cnt + idx) & (RING - 1)
        plsc.store_scatter(ring_s, [pos], zi, mask=m)
        plsc.store_scatter(ring_d, [pos], dumv, mask=m)
        plsc.store_scatter(ring_w, [pos], zf, mask=m)
        return 0
    lax.fori_loop(0, SUPER // 16, pad_body, 0)

    total = cnt + npad
    for _ in range(2):
        cond = flushed < total
        pl.when(cond)(functools.partial(flush, flushed))
        flushed = jnp.where(cond, flushed + FLUSH, flushed)

    nsuper = total >> 11
    cntbuf[pl.ds(0, 16)] = jnp.full((16,), nsuper, jnp.int32)
    pltpu.sync_copy(cntbuf, nsup_h.at[pl.ds(pl.multiple_of((c * NSUB + s) * 16, 8), 16)])


_partition = pl.kernel(
    _partition_body,
    out_type=[
        jax.ShapeDtypeStruct((NTILES * CAP,), jnp.int32),
        jax.ShapeDtypeStruct((NTILES * CAP,), jnp.int32),
        jax.ShapeDtypeStruct((NTILES * CAP,), jnp.float32),
        jax.ShapeDtypeStruct((NTILES * 16,), jnp.int32),
    ],
    mesh=plsc.VectorSubcoreMesh(core_axis_name="c", subcore_axis_name="s"),
    compiler_params=pltpu.CompilerParams(needs_layout_passes=False, use_tc_tiling_on_sc=False),
    scratch_types=[
        pltpu.VMEM((BLK,), jnp.int32),
        pltpu.VMEM((BLK,), jnp.int32),
        pltpu.VMEM((BLK,), jnp.float32),
        pltpu.VMEM((RING,), jnp.int32),
        pltpu.VMEM((RING,), jnp.int32),
        pltpu.VMEM((RING,), jnp.float32),
        pltpu.VMEM((16,), jnp.int32),
    ],
)


def _propagate_body(h_h, esrc_h, edst_h, ew_h, nsup_h, zeros_h, xout_h,
                    acc, ls, ld, lw, rows, rows2, nbuf, semG, semS):
    c = lax.axis_index("c")
    s = lax.axis_index("s")
    lane = lax.iota(jnp.int32, 16)

    pltpu.sync_copy(nsup_h.at[pl.ds(pl.multiple_of((c * NSUB + s) * 16, 8), 16)], nbuf)
    nslots = jnp.max(nbuf[...]) * 8

    pl.when(s < NSUB - 1)(
        lambda: pltpu.sync_copy(zeros_h, acc.at[pl.ds(pl.multiple_of(s * RPT, 8), RPT)]))
    pl.when(s == NSUB - 1)(
        lambda: pltpu.sync_copy(zeros_h.at[pl.ds(0, RPT_LAST)],
                                acc.at[pl.ds((NSUB - 1) * RPT, RPT_LAST)]))
    pl.when(s == 0)(
        lambda: pltpu.sync_copy(zeros_h.at[pl.ds(0, 8)],
                                acc.at[pl.ds(HALF, 8)]))
    plsc.subcore_barrier()

    def slot_body(g, _):
        pltpu.sync_copy(esrc_h.at[c, s, g], ls)
        pltpu.sync_copy(edst_h.at[c, s, g], ld)
        pltpu.sync_copy(ew_h.at[c, s, g], lw)
        for j in range(SLOT_CH):
            pltpu.async_copy(h_h.at[ls.at[j]], rows.at[j], semG)
        for j in range(SLOT_CH):
            pltpu.make_async_copy(h_h.at[ls.at[j]], rows.at[j], semG).wait()

        def grp_body(t, _):
            jch = jnp.full((16,), t >> 3, jnp.int32)
            ev = (t & 7) * 16 + lane
            wv = plsc.load_gather(lw, [jch, ev])
            for col in range(D):
                cv = jnp.full((16,), col, jnp.int32)
                v = plsc.load_gather(rows, [jch, ev, cv])
                plsc.store_scatter(rows2, [jch, ev, cv], v * wv)
            return 0
        lax.fori_loop(0, SLOT_CH * CH // 16, grp_body, 0)

        for j in range(SLOT_CH):
            pltpu.async_copy(rows2.at[j], acc.at[ld.at[j]], semS, add=True)
        for j in range(SLOT_CH):
            pltpu.make_async_copy(rows2.at[j], acc.at[ld.at[j]], semS).wait()
        return 0

    lax.fori_loop(0, nslots, slot_body, 0)
    plsc.subcore_barrier()
    pl.when(s < NSUB - 1)(
        lambda: pltpu.sync_copy(acc.at[pl.ds(pl.multiple_of(s * RPT, 8), RPT)],
                                xout_h.at[pl.ds(pl.multiple_of(c * HALF + s * RPT, 8), RPT)]))
    pl.when(s == NSUB - 1)(
        lambda: pltpu.sync_copy(
            acc.at[pl.ds((NSUB - 1) * RPT, RPT_LAST)],
            xout_h.at[pl.ds(pl.multiple_of(c * HALF + (NSUB - 1) * RPT, 8), RPT_LAST)]))


_propagate = pl.kernel(
    _propagate_body,
    out_type=jax.ShapeDtypeStruct((N_NODES, D), jnp.float32),
    mesh=plsc.VectorSubcoreMesh(core_axis_name="c", subcore_axis_name="s"),
    compiler_params=pltpu.CompilerParams(needs_layout_passes=False, use_tc_tiling_on_sc=False),
    scratch_types=[
        pltpu.VMEM_SHARED((ACC_ROWS, D), jnp.float32),
        pltpu.VMEM((SLOT_CH, CH), jnp.int32),
        pltpu.VMEM((SLOT_CH, CH), jnp.int32),
        pltpu.VMEM((SLOT_CH, CH), jnp.float32),
        pltpu.VMEM((SLOT_CH, CH, D), jnp.float32),
        pltpu.VMEM((SLOT_CH, CH, D), jnp.float32),
        pltpu.VMEM((16,), jnp.int32),
        pltpu.SemaphoreType.DMA,
        pltpu.SemaphoreType.DMA,
    ],
)


def _linear_kernel(x_ref, w_ref, b_ref, o_ref, *, relu):
    x = x_ref[...]
    if relu:
        x = jnp.maximum(x, 0.0)
    o_ref[...] = lax.dot_general(
        x, w_ref[...], (((1,), (1,)), ((), ())),
        preferred_element_type=jnp.float32) + b_ref[...]


def _linear(x, w, b2, relu):
    rb = 4000
    return pl.pallas_call(
        functools.partial(_linear_kernel, relu=relu),
        out_shape=jax.ShapeDtypeStruct((N_NODES, D), jnp.float32),
        grid=(N_NODES // rb,),
        in_specs=[
            pl.BlockSpec((rb, D), lambda i: (i, 0)),
            pl.BlockSpec((D, D), lambda i: (0, 0)),
            pl.BlockSpec((1, D), lambda i: (0, 0)),
        ],
        out_specs=pl.BlockSpec((rb, D), lambda i: (i, 0)),
    )(x, w, b2)


def _relu_kernel(x_ref, o_ref):
    o_ref[...] = jnp.maximum(x_ref[...], 0.0)


def _relu(x):
    rb = 4000
    return pl.pallas_call(
        _relu_kernel,
        out_shape=jax.ShapeDtypeStruct((N_NODES, D), jnp.float32),
        grid=(N_NODES // rb,),
        in_specs=[pl.BlockSpec((rb, D), lambda i: (i, 0))],
        out_specs=pl.BlockSpec((rb, D), lambda i: (i, 0)),
    )(x)


def kernel(edge_index, edge_weight, row_embed, col_embed,
           W0, b0, W1, b1, W2, b2):
    src = edge_index[0].astype(jnp.int32)
    dst = edge_index[1].astype(jnp.int32)
    esrc, edst, ew, nsup = _partition(src, dst, edge_weight)
    esrc5 = esrc.reshape(NCORES, NSUB, NSLOT, SLOT_CH, CH)
    edst5 = edst.reshape(NCORES, NSUB, NSLOT, SLOT_CH, CH)
    ew5 = ew.reshape(NCORES, NSUB, NSLOT, SLOT_CH, CH)
    zeros = jnp.zeros((RPT, D), jnp.float32)
    x = jnp.concatenate([row_embed, col_embed], axis=0)
    for i, (W, b) in enumerate(((W0, b0), (W1, b1), (W2, b2))):
        h = _linear(x, W, b.reshape(1, D), relu=(i > 0))
        x = _propagate(h, esrc5, edst5, ew5, nsup, zeros)
    x = _relu(x)
    return (x[:N_ROWS], x[N_ROWS:])


# speedup vs baseline: 1.8979x; 1.8979x over previous
import functools

import jax
import jax.numpy as jnp
from jax import lax
from jax.experimental import pallas as pl
from jax.experimental.pallas import tpu as pltpu
from jax.experimental.pallas import tpu_sc as plsc

N_ROWS = 50000
N_COLS = 50000
N_NODES = N_ROWS + N_COLS
HALF = 50000
D = 32
N_EDGES = 1600000

NCORES = 2
NSUB = 16
ECHUNK = N_EDGES // NSUB
BLK = 2048
NFULL = ECHUNK // BLK
TAIL = ECHUNK - NFULL * BLK
RING = 4096
FLUSH = 1024
SUPER = 2048
CAP = ((ECHUNK + SUPER - 1) // SUPER + 1) * SUPER
NSLOT = CAP // 256
CH = 128
SLOT_CH = 2
RPT = 3128
RPT_LAST = HALF - (NSUB - 1) * RPT
ACC_ROWS = HALF + 8
DUMMY = HALF
NTILES = NCORES * NSUB


def _partition_body(src_h, dst_h, w_h, esrc_h, edst_h, ew_h, nsup_h,
                    sbuf, dbuf, wbuf, ring_s, ring_d, ring_w, cntbuf):
    c = lax.axis_index("c")
    s = lax.axis_index("s")
    lo = c * HALF
    lane = lax.iota(jnp.int32, 16)
    lbase = (c * NSUB + s) * CAP

    def flush(flushed):
        start = pl.multiple_of(flushed & (RING - 1), 8)
        pltpu.sync_copy(ring_s.at[pl.ds(start, FLUSH)],
                        esrc_h.at[pl.ds(pl.multiple_of(lbase + flushed, 8), FLUSH)])
        pltpu.sync_copy(ring_d.at[pl.ds(start, FLUSH)],
                        edst_h.at[pl.ds(pl.multiple_of(lbase + flushed, 8), FLUSH)])
        pltpu.sync_copy(ring_w.at[pl.ds(start, FLUSH)],
                        ew_h.at[pl.ds(pl.multiple_of(lbase + flushed, 8), FLUSH)])

    def vbody(i, carry):
        cnt, flushed = carry
        off = i * 16
        sv = sbuf[pl.ds(off, 16)]
        dv = dbuf[pl.ds(off, 16)]
        wv = wbuf[pl.ds(off, 16)]
        m = (dv >= lo) & (dv < lo + HALF)
        pc = plsc.cumsum(m.astype(jnp.int32))
        tot = jnp.max(pc)
        pos = (cnt + pc - 1) & (RING - 1)
        plsc.store_scatter(ring_s, [pos], sv, mask=m)
        plsc.store_scatter(ring_d, [pos], dv - lo, mask=m)
        plsc.store_scatter(ring_w, [pos], wv, mask=m)
        cnt = cnt + tot
        cond = (cnt - flushed) >= FLUSH
        pl.when(cond)(lambda: flush(flushed))
        flushed = jnp.where(cond, flushed + FLUSH, flushed)
        return cnt, flushed

    def stage(base, n):
        boff = s * ECHUNK + base
        pltpu.sync_copy(src_h.at[pl.ds(pl.multiple_of(boff, 8), n)], sbuf.at[pl.ds(0, n)])
        pltpu.sync_copy(dst_h.at[pl.ds(pl.multiple_of(boff, 8), n)], dbuf.at[pl.ds(0, n)])
        pltpu.sync_copy(w_h.at[pl.ds(pl.multiple_of(boff, 8), n)], wbuf.at[pl.ds(0, n)])

    def blk_body(b, carry):
        stage(b * BLK, BLK)
        return lax.fori_loop(0, BLK // 16, vbody, carry)

    cnt, flushed = lax.fori_loop(0, NFULL, blk_body,
                                 (jnp.int32(0), jnp.int32(0)))
    stage(NFULL * BLK, TAIL)
    cnt, flushed = lax.fori_loop(0, TAIL // 16, vbody, (cnt, flushed))

    npad = (SUPER - (cnt & (SUPER - 1))) & (SUPER - 1)
    zi = jnp.zeros((16,), jnp.int32)
    zf = jnp.zeros((16,), jnp.float32)
    dumv = jnp.full((16,), DUMMY, jnp.int32)

    def pad_body(i, _):
        idx = i * 16 + lane
        m = idx < npad
        pos = (cnt + idx) & (RING - 1)
        plsc.store_scatter(ring_s, [pos], zi, mask=m)
        plsc.store_scatter(ring_d, [pos], dumv, mask=m)
        plsc.store_scatter(ring_w, [pos], zf, mask=m)
        return 0
    lax.fori_loop(0, SUPER // 16, pad_body, 0)

    total = cnt + npad
    for _ in range(2):
        cond = flushed < total
        pl.when(cond)(functools.partial(flush, flushed))
        flushed = jnp.where(cond, flushed + FLUSH, flushed)

    nsuper = total >> 11
    cntbuf[pl.ds(0, 16)] = jnp.full((16,), nsuper, jnp.int32)
    pltpu.sync_copy(cntbuf, nsup_h.at[pl.ds(pl.multiple_of((c * NSUB + s) * 16, 8), 16)])


_partition = pl.kernel(
    _partition_body,
    out_type=[
        jax.ShapeDtypeStruct((NTILES * CAP,), jnp.int32),
        jax.ShapeDtypeStruct((NTILES * CAP,), jnp.int32),
        jax.ShapeDtypeStruct((NTILES * CAP,), jnp.float32),
        jax.ShapeDtypeStruct((NTILES * 16,), jnp.int32),
    ],
    mesh=plsc.VectorSubcoreMesh(core_axis_name="c", subcore_axis_name="s"),
    compiler_params=pltpu.CompilerParams(needs_layout_passes=False, use_tc_tiling_on_sc=False),
    scratch_types=[
        pltpu.VMEM((BLK,), jnp.int32),
        pltpu.VMEM((BLK,), jnp.int32),
        pltpu.VMEM((BLK,), jnp.float32),
        pltpu.VMEM((RING,), jnp.int32),
        pltpu.VMEM((RING,), jnp.int32),
        pltpu.VMEM((RING,), jnp.float32),
        pltpu.VMEM((16,), jnp.int32),
    ],
)


def _propagate_body(h_h, esrc_h, edst_h, ew_h, nsup_h, zeros_h, xout_h,
                    acc, ls, ld, lw, rows, rows2, nbuf, semG, semS):
    c = lax.axis_index("c")
    s = lax.axis_index("s")
    lane = lax.iota(jnp.int32, 16)

    pltpu.sync_copy(nsup_h.at[pl.ds(pl.multiple_of((c * NSUB + s) * 16, 8), 16)], nbuf)
    nslots = jnp.max(nbuf[...]) * 8

    pl.when(s < NSUB - 1)(
        lambda: pltpu.sync_copy(zeros_h, acc.at[pl.ds(pl.multiple_of(s * RPT, 8), RPT)]))
    pl.when(s == NSUB - 1)(
        lambda: pltpu.sync_copy(zeros_h.at[pl.ds(0, RPT_LAST)],
                                acc.at[pl.ds((NSUB - 1) * RPT, RPT_LAST)]))
    pl.when(s == 0)(
        lambda: pltpu.sync_copy(zeros_h.at[pl.ds(0, 8)],
                                acc.at[pl.ds(HALF, 8)]))
    plsc.subcore_barrier()

    def slot_body(g, _):
        pltpu.sync_copy(esrc_h.at[c, s, g], ls)
        pltpu.sync_copy(edst_h.at[c, s, g], ld)
        pltpu.sync_copy(ew_h.at[c, s, g], lw)
        for j in range(SLOT_CH):
            pltpu.async_copy(h_h.at[ls.at[j]], rows.at[j], semG)
        for j in range(SLOT_CH):
            pltpu.make_async_copy(h_h.at[ls.at[j]], rows.at[j], semG).wait()

        def grp_body(t, _):
            jch = jnp.full((16,), t >> 3, jnp.int32)
            ev = (t & 7) * 16 + lane
            wv = plsc.load_gather(lw, [jch, ev])
            for col in range(D):
                cv = (lane + col) & (D - 1)
                v = plsc.load_gather(rows, [jch, ev, cv])
                plsc.store_scatter(rows2, [jch, ev, cv], v * wv)
            return 0
        lax.fori_loop(0, SLOT_CH * CH // 16, grp_body, 0)

        for j in range(SLOT_CH):
            pltpu.async_copy(rows2.at[j], acc.at[ld.at[j]], semS, add=True)
        for j in range(SLOT_CH):
            pltpu.make_async_copy(rows2.at[j], acc.at[ld.at[j]], semS).wait()
        return 0

    lax.fori_loop(0, nslots, slot_body, 0)
    plsc.subcore_barrier()
    pl.when(s < NSUB - 1)(
        lambda: pltpu.sync_copy(acc.at[pl.ds(pl.multiple_of(s * RPT, 8), RPT)],
                                xout_h.at[pl.ds(pl.multiple_of(c * HALF + s * RPT, 8), RPT)]))
    pl.when(s == NSUB - 1)(
        lambda: pltpu.sync_copy(
            acc.at[pl.ds((NSUB - 1) * RPT, RPT_LAST)],
            xout_h.at[pl.ds(pl.multiple_of(c * HALF + (NSUB - 1) * RPT, 8), RPT_LAST)]))


_propagate = pl.kernel(
    _propagate_body,
    out_type=jax.ShapeDtypeStruct((N_NODES, D), jnp.float32),
    mesh=plsc.VectorSubcoreMesh(core_axis_name="c", subcore_axis_name="s"),
    compiler_params=pltpu.CompilerParams(needs_layout_passes=False, use_tc_tiling_on_sc=False),
    scratch_types=[
        pltpu.VMEM_SHARED((ACC_ROWS, D), jnp.float32),
        pltpu.VMEM((SLOT_CH, CH), jnp.int32),
        pltpu.VMEM((SLOT_CH, CH), jnp.int32),
        pltpu.VMEM((SLOT_CH, CH), jnp.float32),
        pltpu.VMEM((SLOT_CH, CH, D), jnp.float32),
        pltpu.VMEM((SLOT_CH, CH, D), jnp.float32),
        pltpu.VMEM((16,), jnp.int32),
        pltpu.SemaphoreType.DMA,
        pltpu.SemaphoreType.DMA,
    ],
)


def _linear_kernel(x_ref, w_ref, b_ref, o_ref, *, relu):
    x = x_ref[...]
    if relu:
        x = jnp.maximum(x, 0.0)
    o_ref[...] = lax.dot_general(
        x, w_ref[...], (((1,), (1,)), ((), ())),
        preferred_element_type=jnp.float32) + b_ref[...]


def _linear(x, w, b2, relu):
    rb = 4000
    return pl.pallas_call(
        functools.partial(_linear_kernel, relu=relu),
        out_shape=jax.ShapeDtypeStruct((N_NODES, D), jnp.float32),
        grid=(N_NODES // rb,),
        in_specs=[
            pl.BlockSpec((rb, D), lambda i: (i, 0)),
            pl.BlockSpec((D, D), lambda i: (0, 0)),
            pl.BlockSpec((1, D), lambda i: (0, 0)),
        ],
        out_specs=pl.BlockSpec((rb, D), lambda i: (i, 0)),
    )(x, w, b2)


def _relu_kernel(x_ref, o_ref):
    o_ref[...] = jnp.maximum(x_ref[...], 0.0)


def _relu(x):
    rb = 4000
    return pl.pallas_call(
        _relu_kernel,
        out_shape=jax.ShapeDtypeStruct((N_NODES, D), jnp.float32),
        grid=(N_NODES // rb,),
        in_specs=[pl.BlockSpec((rb, D), lambda i: (i, 0))],
        out_specs=pl.BlockSpec((rb, D), lambda i: (i, 0)),
    )(x)


def kernel(edge_index, edge_weight, row_embed, col_embed,
           W0, b0, W1, b1, W2, b2):
    src = edge_index[0].astype(jnp.int32)
    dst = edge_index[1].astype(jnp.int32)
    esrc, edst, ew, nsup = _partition(src, dst, edge_weight)
    esrc5 = esrc.reshape(NCORES, NSUB, NSLOT, SLOT_CH, CH)
    edst5 = edst.reshape(NCORES, NSUB, NSLOT, SLOT_CH, CH)
    ew5 = ew.reshape(NCORES, NSUB, NSLOT, SLOT_CH, CH)
    zeros = jnp.zeros((RPT, D), jnp.float32)
    x = jnp.concatenate([row_embed, col_embed], axis=0)
    for i, (W, b) in enumerate(((W0, b0), (W1, b1), (W2, b2))):
        h = _linear(x, W, b.reshape(1, D), relu=(i > 0))
        x = _propagate(h, esrc5, edst5, ew5, nsup, zeros)
    x = _relu(x)
    return (x[:N_ROWS], x[N_ROWS:])


# speedup vs baseline: 1.9001x; 1.0011x over previous
import functools

import jax
import jax.numpy as jnp
from jax import lax
from jax.experimental import pallas as pl
from jax.experimental.pallas import tpu as pltpu
from jax.experimental.pallas import tpu_sc as plsc

N_ROWS = 50000
N_COLS = 50000
N_NODES = N_ROWS + N_COLS
HALF = 50000
D = 32
N_EDGES = 1600000

NCORES = 2
NSUB = 16
ECHUNK = N_EDGES // NSUB
BLK = 2048
NFULL = ECHUNK // BLK
TAIL = ECHUNK - NFULL * BLK
RING = 4096
FLUSH = 1024
SUPER = 2048
CAP = ((ECHUNK + SUPER - 1) // SUPER + 1) * SUPER
CH = 128
NSLOT = CAP // CH
DEPTH = 4
RPT = 3128
RPT_LAST = HALF - (NSUB - 1) * RPT
ACC_ROWS = HALF + 8
DUMMY = HALF
NTILES = NCORES * NSUB


def _partition_body(src_h, dst_h, w_h, esrc_h, edst_h, ew_h, nsup_h,
                    sbuf, dbuf, wbuf, ring_s, ring_d, ring_w, cntbuf):
    c = lax.axis_index("c")
    s = lax.axis_index("s")
    lo = c * HALF
    lane = lax.iota(jnp.int32, 16)
    lbase = (c * NSUB + s) * CAP

    def flush(flushed):
        start = pl.multiple_of(flushed & (RING - 1), 8)
        pltpu.sync_copy(ring_s.at[pl.ds(start, FLUSH)],
                        esrc_h.at[pl.ds(pl.multiple_of(lbase + flushed, 8), FLUSH)])
        pltpu.sync_copy(ring_d.at[pl.ds(start, FLUSH)],
                        edst_h.at[pl.ds(pl.multiple_of(lbase + flushed, 8), FLUSH)])
        pltpu.sync_copy(ring_w.at[pl.ds(start, FLUSH)],
                        ew_h.at[pl.ds(pl.multiple_of(lbase + flushed, 8), FLUSH)])

    def vbody(i, carry):
        cnt, flushed = carry
        off = i * 16
        sv = sbuf[pl.ds(off, 16)]
        dv = dbuf[pl.ds(off, 16)]
        wv = wbuf[pl.ds(off, 16)]
        m = (dv >= lo) & (dv < lo + HALF)
        pc = plsc.cumsum(m.astype(jnp.int32))
        tot = jnp.max(pc)
        pos = (cnt + pc - 1) & (RING - 1)
        plsc.store_scatter(ring_s, [pos], sv, mask=m)
        plsc.store_scatter(ring_d, [pos], dv - lo, mask=m)
        plsc.store_scatter(ring_w, [pos], wv, mask=m)
        cnt = cnt + tot
        cond = (cnt - flushed) >= FLUSH
        pl.when(cond)(lambda: flush(flushed))
        flushed = jnp.where(cond, flushed + FLUSH, flushed)
        return cnt, flushed

    def stage(base, n):
        boff = s * ECHUNK + base
        pltpu.sync_copy(src_h.at[pl.ds(pl.multiple_of(boff, 8), n)], sbuf.at[pl.ds(0, n)])
        pltpu.sync_copy(dst_h.at[pl.ds(pl.multiple_of(boff, 8), n)], dbuf.at[pl.ds(0, n)])
        pltpu.sync_copy(w_h.at[pl.ds(pl.multiple_of(boff, 8), n)], wbuf.at[pl.ds(0, n)])

    def blk_body(b, carry):
        stage(b * BLK, BLK)
        return lax.fori_loop(0, BLK // 16, vbody, carry)

    cnt, flushed = lax.fori_loop(0, NFULL, blk_body,
                                 (jnp.int32(0), jnp.int32(0)))
    stage(NFULL * BLK, TAIL)
    cnt, flushed = lax.fori_loop(0, TAIL // 16, vbody, (cnt, flushed))

    npad = (SUPER - (cnt & (SUPER - 1))) & (SUPER - 1)
    zi = jnp.zeros((16,), jnp.int32)
    zf = jnp.zeros((16,), jnp.float32)
    dumv = jnp.full((16,), DUMMY, jnp.int32)

    def pad_body(i, _):
        idx = i * 16 + lane
        m = idx < npad
        pos = (cnt + idx) & (RING - 1)
        plsc.store_scatter(ring_s, [pos], zi, mask=m)
        plsc.store_scatter(ring_d, [pos], dumv, mask=m)
        plsc.store_scatter(ring_w, [pos], zf, mask=m)
        return 0
    lax.fori_loop(0, SUPER // 16, pad_body, 0)

    total = cnt + npad
    for _ in range(2):
        cond = flushed < total
        pl.when(cond)(functools.partial(flush, flushed))
        flushed = jnp.where(cond, flushed + FLUSH, flushed)

    nsuper = total >> 11
    cntbuf[pl.ds(0, 16)] = jnp.full((16,), nsuper, jnp.int32)
    pltpu.sync_copy(cntbuf, nsup_h.at[pl.ds(pl.multiple_of((c * NSUB + s) * 16, 8), 16)])


_partition = pl.kernel(
    _partition_body,
    out_type=[
        jax.ShapeDtypeStruct((NTILES * CAP,), jnp.int32),
        jax.ShapeDtypeStruct((NTILES * CAP,), jnp.int32),
        jax.ShapeDtypeStruct((NTILES * CAP,), jnp.float32),
        jax.ShapeDtypeStruct((NTILES * 16,), jnp.int32),
    ],
    mesh=plsc.VectorSubcoreMesh(core_axis_name="c", subcore_axis_name="s"),
    compiler_params=pltpu.CompilerParams(needs_layout_passes=False, use_tc_tiling_on_sc=False),
    scratch_types=[
        pltpu.VMEM((BLK,), jnp.int32),
        pltpu.VMEM((BLK,), jnp.int32),
        pltpu.VMEM((BLK,), jnp.float32),
        pltpu.VMEM((RING,), jnp.int32),
        pltpu.VMEM((RING,), jnp.int32),
        pltpu.VMEM((RING,), jnp.float32),
        pltpu.VMEM((16,), jnp.int32),
    ],
)


def _propagate_body(h_h, esrc_h, edst_h, ew_h, nsup_h, zeros_h, xout_h,
                    acc, ls, ld, lw, rows, nbuf,
                    sg0, sg1, sg2, sg3, ss0, ss1, ss2, ss3):
    c = lax.axis_index("c")
    s = lax.axis_index("s")
    lane = lax.iota(jnp.int32, 16)
    semG = (sg0, sg1, sg2, sg3)
    semS = (ss0, ss1, ss2, ss3)

    pltpu.sync_copy(nsup_h.at[pl.ds(pl.multiple_of((c * NSUB + s) * 16, 8), 16)], nbuf)
    nslots = jnp.max(nbuf[...]) * (SUPER // CH)

    pl.when(s < NSUB - 1)(
        lambda: pltpu.sync_copy(zeros_h, acc.at[pl.ds(pl.multiple_of(s * RPT, 8), RPT)]))
    pl.when(s == NSUB - 1)(
        lambda: pltpu.sync_copy(zeros_h.at[pl.ds(0, RPT_LAST)],
                                acc.at[pl.ds((NSUB - 1) * RPT, RPT_LAST)]))
    pl.when(s == 0)(
        lambda: pltpu.sync_copy(zeros_h.at[pl.ds(0, 8)],
                                acc.at[pl.ds(HALF, 8)]))
    plsc.subcore_barrier()

    def prep(g, q):
        pltpu.sync_copy(esrc_h.at[c, s, g], ls.at[q])
        pltpu.sync_copy(edst_h.at[c, s, g], ld.at[q])
        pltpu.sync_copy(ew_h.at[c, s, g], lw.at[q])
        pltpu.async_copy(h_h.at[ls.at[q]], rows.at[q], semG[q])

    for k in range(2):
        pl.when(k < nslots)(functools.partial(prep, k, k))

    def sub(g, q):
        r = (q + 2) & (DEPTH - 1)
        pltpu.make_async_copy(h_h.at[ls.at[q]], rows.at[q], semG[q]).wait()
        qv = jnp.full((16,), q, jnp.int32)

        def grp_body(t, _):
            ev = t * 16 + lane
            wv = plsc.load_gather(lw, [qv, ev])
            for col in range(D):
                cv = (lane + col) & (D - 1)
                v = plsc.load_gather(rows, [qv, ev, cv])
                plsc.store_scatter(rows, [qv, ev, cv], v * wv)
            return 0
        lax.fori_loop(0, CH // 16, grp_body, 0)

        pltpu.async_copy(rows.at[q], acc.at[ld.at[q]], semS[q], add=True)
        nxt = g + 2
        pl.when((nxt < nslots) & (g >= 2))(
            lambda: pltpu.make_async_copy(rows.at[r], acc.at[ld.at[r]], semS[r]).wait())
        pl.when(nxt < nslots)(lambda: prep(nxt, r))
        return 0

    def body4(i, _):
        g = i * DEPTH
        for q in range(DEPTH):
            sub(g + q, q)
        return 0

    lax.fori_loop(0, nslots // DEPTH, body4, 0)
    for q in range(DEPTH):
        pl.when(q < nslots)(
            functools.partial(
                lambda qq: pltpu.make_async_copy(
                    rows.at[qq], acc.at[ld.at[qq]], semS[qq]).wait(), q))
    plsc.subcore_barrier()
    pl.when(s < NSUB - 1)(
        lambda: pltpu.sync_copy(acc.at[pl.ds(pl.multiple_of(s * RPT, 8), RPT)],
                                xout_h.at[pl.ds(pl.multiple_of(c * HALF + s * RPT, 8), RPT)]))
    pl.when(s == NSUB - 1)(
        lambda: pltpu.sync_copy(
            acc.at[pl.ds((NSUB - 1) * RPT, RPT_LAST)],
            xout_h.at[pl.ds(pl.multiple_of(c * HALF + (NSUB - 1) * RPT, 8), RPT_LAST)]))


_propagate = pl.kernel(
    _propagate_body,
    out_type=jax.ShapeDtypeStruct((N_NODES, D), jnp.float32),
    mesh=plsc.VectorSubcoreMesh(core_axis_name="c", subcore_axis_name="s"),
    compiler_params=pltpu.CompilerParams(needs_layout_passes=False, use_tc_tiling_on_sc=False),
    scratch_types=[
        pltpu.VMEM_SHARED((ACC_ROWS, D), jnp.float32),
        pltpu.VMEM((DEPTH, CH), jnp.int32),
        pltpu.VMEM((DEPTH, CH), jnp.int32),
        pltpu.VMEM((DEPTH, CH), jnp.float32),
        pltpu.VMEM((DEPTH, CH, D), jnp.float32),
        pltpu.VMEM((16,), jnp.int32),
    ] + [pltpu.SemaphoreType.DMA] * (2 * DEPTH),
)


def _linear_kernel(x_ref, w_ref, b_ref, o_ref, *, relu):
    x = x_ref[...]
    if relu:
        x = jnp.maximum(x, 0.0)
    o_ref[...] = lax.dot_general(
        x, w_ref[...], (((1,), (1,)), ((), ())),
        preferred_element_type=jnp.float32) + b_ref[...]


def _linear(x, w, b2, relu):
    rb = 4000
    return pl.pallas_call(
        functools.partial(_linear_kernel, relu=relu),
        out_shape=jax.ShapeDtypeStruct((N_NODES, D), jnp.float32),
        grid=(N_NODES // rb,),
        in_specs=[
            pl.BlockSpec((rb, D), lambda i: (i, 0)),
            pl.BlockSpec((D, D), lambda i: (0, 0)),
            pl.BlockSpec((1, D), lambda i: (0, 0)),
        ],
        out_specs=pl.BlockSpec((rb, D), lambda i: (i, 0)),
    )(x, w, b2)


def _relu_kernel(x_ref, o_ref):
    o_ref[...] = jnp.maximum(x_ref[...], 0.0)


def _relu(x):
    rb = 4000
    return pl.pallas_call(
        _relu_kernel,
        out_shape=jax.ShapeDtypeStruct((N_NODES, D), jnp.float32),
        grid=(N_NODES // rb,),
        in_specs=[pl.BlockSpec((rb, D), lambda i: (i, 0))],
        out_specs=pl.BlockSpec((rb, D), lambda i: (i, 0)),
    )(x)


def kernel(edge_index, edge_weight, row_embed, col_embed,
           W0, b0, W1, b1, W2, b2):
    src = edge_index[0].astype(jnp.int32)
    dst = edge_index[1].astype(jnp.int32)
    esrc, edst, ew, nsup = _partition(src, dst, edge_weight)
    esrc5 = esrc.reshape(NCORES, NSUB, NSLOT, CH)
    edst5 = edst.reshape(NCORES, NSUB, NSLOT, CH)
    ew5 = ew.reshape(NCORES, NSUB, NSLOT, CH)
    zeros = jnp.zeros((RPT, D), jnp.float32)
    x = jnp.concatenate([row_embed, col_embed], axis=0)
    for i, (W, b) in enumerate(((W0, b0), (W1, b1), (W2, b2))):
        h = _linear(x, W, b.reshape(1, D), relu=(i > 0))
        x = _propagate(h, esrc5, edst5, ew5, nsup, zeros)
    x = _relu(x)
    return (x[:N_ROWS], x[N_ROWS:])


# speedup vs baseline: 2.5065x; 1.3191x over previous
import functools

import jax
import jax.numpy as jnp
from jax import lax
from jax.experimental import pallas as pl
from jax.experimental.pallas import tpu as pltpu
from jax.experimental.pallas import tpu_sc as plsc

N_ROWS = 50000
N_COLS = 50000
N_NODES = N_ROWS + N_COLS
HALF = 50000
D = 32
N_EDGES = 1600000

NCORES = 2
NSUB = 16
ECHUNK = N_EDGES // NSUB
BLK = 2048
NFULL = ECHUNK // BLK
TAIL = ECHUNK - NFULL * BLK
RING = 4096
FLUSH = 1024
SUPER = 2048
CAP = ((ECHUNK + SUPER - 1) // SUPER + 1) * SUPER
CH = 128
NSLOT = CAP // CH
DEPTH = 4
RPT = 3128
RPT_LAST = HALF - (NSUB - 1) * RPT
ACC_ROWS = HALF + 8
DUMMY = HALF
NTILES = NCORES * NSUB


def _partition_body(src_h, dst_h, w_h, esrc_h, edst_h, ew_h, nsup_h,
                    sbuf, dbuf, wbuf, ring_s, ring_d, ring_w, cntbuf):
    c = lax.axis_index("c")
    s = lax.axis_index("s")
    lo = c * HALF
    lane = lax.iota(jnp.int32, 16)
    lbase = (c * NSUB + s) * CAP

    def flush(flushed):
        start = pl.multiple_of(flushed & (RING - 1), 8)
        pltpu.sync_copy(ring_s.at[pl.ds(start, FLUSH)],
                        esrc_h.at[pl.ds(pl.multiple_of(lbase + flushed, 8), FLUSH)])
        pltpu.sync_copy(ring_d.at[pl.ds(start, FLUSH)],
                        edst_h.at[pl.ds(pl.multiple_of(lbase + flushed, 8), FLUSH)])
        pltpu.sync_copy(ring_w.at[pl.ds(start, FLUSH)],
                        ew_h.at[pl.ds(pl.multiple_of(lbase + flushed, 8), FLUSH)])

    def vbody(i, carry):
        cnt, flushed = carry
        off = i * 16
        sv = sbuf[pl.ds(off, 16)]
        dv = dbuf[pl.ds(off, 16)]
        wv = wbuf[pl.ds(off, 16)]
        m = (dv >= lo) & (dv < lo + HALF)
        pc = plsc.cumsum(m.astype(jnp.int32))
        tot = jnp.max(pc)
        pos = (cnt + pc - 1) & (RING - 1)
        plsc.store_scatter(ring_s, [pos], sv, mask=m)
        plsc.store_scatter(ring_d, [pos], dv - lo, mask=m)
        plsc.store_scatter(ring_w, [pos], wv, mask=m)
        cnt = cnt + tot
        cond = (cnt - flushed) >= FLUSH
        pl.when(cond)(lambda: flush(flushed))
        flushed = jnp.where(cond, flushed + FLUSH, flushed)
        return cnt, flushed

    def stage(base, n):
        boff = s * ECHUNK + base
        pltpu.sync_copy(src_h.at[pl.ds(pl.multiple_of(boff, 8), n)], sbuf.at[pl.ds(0, n)])
        pltpu.sync_copy(dst_h.at[pl.ds(pl.multiple_of(boff, 8), n)], dbuf.at[pl.ds(0, n)])
        pltpu.sync_copy(w_h.at[pl.ds(pl.multiple_of(boff, 8), n)], wbuf.at[pl.ds(0, n)])

    def blk_body(b, carry):
        stage(b * BLK, BLK)
        return lax.fori_loop(0, BLK // 16, vbody, carry)

    cnt, flushed = lax.fori_loop(0, NFULL, blk_body,
                                 (jnp.int32(0), jnp.int32(0)))
    stage(NFULL * BLK, TAIL)
    cnt, flushed = lax.fori_loop(0, TAIL // 16, vbody, (cnt, flushed))

    npad = (SUPER - (cnt & (SUPER - 1))) & (SUPER - 1)
    zi = jnp.zeros((16,), jnp.int32)
    zf = jnp.zeros((16,), jnp.float32)
    dumv = jnp.full((16,), DUMMY, jnp.int32)

    def pad_body(i, _):
        idx = i * 16 + lane
        m = idx < npad
        pos = (cnt + idx) & (RING - 1)
        plsc.store_scatter(ring_s, [pos], zi, mask=m)
        plsc.store_scatter(ring_d, [pos], dumv, mask=m)
        plsc.store_scatter(ring_w, [pos], zf, mask=m)
        return 0
    lax.fori_loop(0, SUPER // 16, pad_body, 0)

    total = cnt + npad
    for _ in range(2):
        cond = flushed < total
        pl.when(cond)(functools.partial(flush, flushed))
        flushed = jnp.where(cond, flushed + FLUSH, flushed)

    nsuper = total >> 11
    cntbuf[pl.ds(0, 16)] = jnp.full((16,), nsuper, jnp.int32)
    pltpu.sync_copy(cntbuf, nsup_h.at[pl.ds(pl.multiple_of((c * NSUB + s) * 16, 8), 16)])


_partition = pl.kernel(
    _partition_body,
    out_type=[
        jax.ShapeDtypeStruct((NTILES * CAP,), jnp.int32),
        jax.ShapeDtypeStruct((NTILES * CAP,), jnp.int32),
        jax.ShapeDtypeStruct((NTILES * CAP,), jnp.float32),
        jax.ShapeDtypeStruct((NTILES * 16,), jnp.int32),
    ],
    mesh=plsc.VectorSubcoreMesh(core_axis_name="c", subcore_axis_name="s"),
    compiler_params=pltpu.CompilerParams(needs_layout_passes=False, use_tc_tiling_on_sc=False),
    scratch_types=[
        pltpu.VMEM((BLK,), jnp.int32),
        pltpu.VMEM((BLK,), jnp.int32),
        pltpu.VMEM((BLK,), jnp.float32),
        pltpu.VMEM((RING,), jnp.int32),
        pltpu.VMEM((RING,), jnp.int32),
        pltpu.VMEM((RING,), jnp.float32),
        pltpu.VMEM((16,), jnp.int32),
    ],
)


def _propagate_body(h_h, esrc_h, edst_h, ew_h, nsup_h, zeros_h, xout_h,
                    acc, ls, ld, lw, rows, nbuf,
                    sg0, sg1, sg2, sg3, ss0, ss1, ss2, ss3):
    c = lax.axis_index("c")
    s = lax.axis_index("s")
    lane = lax.iota(jnp.int32, 16)
    semG = (sg0, sg1, sg2, sg3)
    semS = (ss0, ss1, ss2, ss3)

    pltpu.sync_copy(nsup_h.at[pl.ds(pl.multiple_of((c * NSUB + s) * 16, 8), 16)], nbuf)
    nslots = jnp.max(nbuf[...]) * (SUPER // CH)

    pl.when(s < NSUB - 1)(
        lambda: pltpu.sync_copy(zeros_h, acc.at[pl.ds(pl.multiple_of(s * RPT, 8), RPT)]))
    pl.when(s == NSUB - 1)(
        lambda: pltpu.sync_copy(zeros_h.at[pl.ds(0, RPT_LAST)],
                                acc.at[pl.ds((NSUB - 1) * RPT, RPT_LAST)]))
    pl.when(s == 0)(
        lambda: pltpu.sync_copy(zeros_h.at[pl.ds(0, 8)],
                                acc.at[pl.ds(HALF, 8)]))
    plsc.subcore_barrier()

    def prep(g, q):
        pltpu.sync_copy(esrc_h.at[c, s, g], ls.at[q])
        pltpu.sync_copy(edst_h.at[c, s, g], ld.at[q])
        pltpu.sync_copy(ew_h.at[c, s, g], lw.at[q])
        pltpu.async_copy(h_h.at[ls.at[q]], rows.at[q], semG[q])

    for k in range(2):
        pl.when(k < nslots)(functools.partial(prep, k, k))

    def sub(g, q):
        r = (q + 2) & (DEPTH - 1)
        pltpu.make_async_copy(h_h.at[ls.at[q]], rows.at[q], semG[q]).wait()
        def grp_body(t, _):
            wv16 = lw[q, pl.ds(t * 16, 16)]
            for u in range(16):
                e = t * 16 + u
                w = wv16[u]
                rows[q, e, pl.ds(0, 16)] = rows[q, e, pl.ds(0, 16)] * w
                rows[q, e, pl.ds(16, 16)] = rows[q, e, pl.ds(16, 16)] * w
            return 0
        lax.fori_loop(0, CH // 16, grp_body, 0)

        pltpu.async_copy(rows.at[q], acc.at[ld.at[q]], semS[q], add=True)
        nxt = g + 2
        pl.when((nxt < nslots) & (g >= 2))(
            lambda: pltpu.make_async_copy(rows.at[r], acc.at[ld.at[r]], semS[r]).wait())
        pl.when(nxt < nslots)(lambda: prep(nxt, r))
        return 0

    def body4(i, _):
        g = i * DEPTH
        for q in range(DEPTH):
            sub(g + q, q)
        return 0

    lax.fori_loop(0, nslots // DEPTH, body4, 0)
    for q in range(DEPTH):
        pl.when(q < nslots)(
            functools.partial(
                lambda qq: pltpu.make_async_copy(
                    rows.at[qq], acc.at[ld.at[qq]], semS[qq]).wait(), q))
    plsc.subcore_barrier()
    pl.when(s < NSUB - 1)(
        lambda: pltpu.sync_copy(acc.at[pl.ds(pl.multiple_of(s * RPT, 8), RPT)],
                                xout_h.at[pl.ds(pl.multiple_of(c * HALF + s * RPT, 8), RPT)]))
    pl.when(s == NSUB - 1)(
        lambda: pltpu.sync_copy(
            acc.at[pl.ds((NSUB - 1) * RPT, RPT_LAST)],
            xout_h.at[pl.ds(pl.multiple_of(c * HALF + (NSUB - 1) * RPT, 8), RPT_LAST)]))


_propagate = pl.kernel(
    _propagate_body,
    out_type=jax.ShapeDtypeStruct((N_NODES, D), jnp.float32),
    mesh=plsc.VectorSubcoreMesh(core_axis_name="c", subcore_axis_name="s"),
    compiler_params=pltpu.CompilerParams(needs_layout_passes=False, use_tc_tiling_on_sc=False),
    scratch_types=[
        pltpu.VMEM_SHARED((ACC_ROWS, D), jnp.float32),
        pltpu.VMEM((DEPTH, CH), jnp.int32),
        pltpu.VMEM((DEPTH, CH), jnp.int32),
        pltpu.VMEM((DEPTH, CH), jnp.float32),
        pltpu.VMEM((DEPTH, CH, D), jnp.float32),
        pltpu.VMEM((16,), jnp.int32),
    ] + [pltpu.SemaphoreType.DMA] * (2 * DEPTH),
)


def _linear_kernel(x_ref, w_ref, b_ref, o_ref, *, relu):
    x = x_ref[...]
    if relu:
        x = jnp.maximum(x, 0.0)
    o_ref[...] = lax.dot_general(
        x, w_ref[...], (((1,), (1,)), ((), ())),
        preferred_element_type=jnp.float32) + b_ref[...]


def _linear(x, w, b2, relu):
    rb = 4000
    return pl.pallas_call(
        functools.partial(_linear_kernel, relu=relu),
        out_shape=jax.ShapeDtypeStruct((N_NODES, D), jnp.float32),
        grid=(N_NODES // rb,),
        in_specs=[
            pl.BlockSpec((rb, D), lambda i: (i, 0)),
            pl.BlockSpec((D, D), lambda i: (0, 0)),
            pl.BlockSpec((1, D), lambda i: (0, 0)),
        ],
        out_specs=pl.BlockSpec((rb, D), lambda i: (i, 0)),
    )(x, w, b2)


def _relu_kernel(x_ref, o_ref):
    o_ref[...] = jnp.maximum(x_ref[...], 0.0)


def _relu(x):
    rb = 4000
    return pl.pallas_call(
        _relu_kernel,
        out_shape=jax.ShapeDtypeStruct((N_NODES, D), jnp.float32),
        grid=(N_NODES // rb,),
        in_specs=[pl.BlockSpec((rb, D), lambda i: (i, 0))],
        out_specs=pl.BlockSpec((rb, D), lambda i: (i, 0)),
    )(x)


def kernel(edge_index, edge_weight, row_embed, col_embed,
           W0, b0, W1, b1, W2, b2):
    src = edge_index[0].astype(jnp.int32)
    dst = edge_index[1].astype(jnp.int32)
    esrc, edst, ew, nsup = _partition(src, dst, edge_weight)
    esrc5 = esrc.reshape(NCORES, NSUB, NSLOT, CH)
    edst5 = edst.reshape(NCORES, NSUB, NSLOT, CH)
    ew5 = ew.reshape(NCORES, NSUB, NSLOT, CH)
    zeros = jnp.zeros((RPT, D), jnp.float32)
    x = jnp.concatenate([row_embed, col_embed], axis=0)
    for i, (W, b) in enumerate(((W0, b0), (W1, b1), (W2, b2))):
        h = _linear(x, W, b.reshape(1, D), relu=(i > 0))
        x = _propagate(h, esrc5, edst5, ew5, nsup, zeros)
    x = _relu(x)
    return (x[:N_ROWS], x[N_ROWS:])


# speedup vs baseline: 3.3764x; 1.3471x over previous
import functools

import jax
import jax.numpy as jnp
from jax import lax
from jax.experimental import pallas as pl
from jax.experimental.pallas import tpu as pltpu
from jax.experimental.pallas import tpu_sc as plsc

N_ROWS = 50000
N_COLS = 50000
N_NODES = N_ROWS + N_COLS
HALF = 50000
D = 32
N_EDGES = 1600000

NCORES = 2
NSUB = 16
ECHUNK = N_EDGES // NSUB
BLK = 2048
NFULL = ECHUNK // BLK
TAIL = ECHUNK - NFULL * BLK
RING = 4096
FLUSH = 1024
SUPER = 2048
CAP = ((ECHUNK + SUPER - 1) // SUPER + 1) * SUPER
CH = 128
NSLOT = CAP // CH
DEPTH = 4
RPT = 3128
RPT_LAST = HALF - (NSUB - 1) * RPT
ACC_ROWS = HALF + 8
DUMMY = HALF
NTILES = NCORES * NSUB


def _partition_body(src_h, dst_h, w_h, esrc_h, edst_h, ew_h, nsup_h,
                    sbuf, dbuf, wbuf, ring_s, ring_d, ring_w, cntbuf):
    c = lax.axis_index("c")
    s = lax.axis_index("s")
    lo = c * HALF
    lane = lax.iota(jnp.int32, 16)
    lbase = (c * NSUB + s) * CAP

    def flush(flushed):
        start = pl.multiple_of(flushed & (RING - 1), 8)
        pltpu.sync_copy(ring_s.at[pl.ds(start, FLUSH)],
                        esrc_h.at[pl.ds(pl.multiple_of(lbase + flushed, 8), FLUSH)])
        pltpu.sync_copy(ring_d.at[pl.ds(start, FLUSH)],
                        edst_h.at[pl.ds(pl.multiple_of(lbase + flushed, 8), FLUSH)])
        pltpu.sync_copy(ring_w.at[pl.ds(start, FLUSH)],
                        ew_h.at[pl.ds(pl.multiple_of(lbase + flushed, 8), FLUSH)])

    def vbody(i, carry):
        cnt, flushed = carry
        off = i * 16
        sv = sbuf[pl.ds(off, 16)]
        dv = dbuf[pl.ds(off, 16)]
        wv = wbuf[pl.ds(off, 16)]
        m = (dv >= lo) & (dv < lo + HALF)
        pc = plsc.cumsum(m.astype(jnp.int32))
        tot = jnp.max(pc)
        pos = (cnt + pc - 1) & (RING - 1)
        plsc.store_scatter(ring_s, [pos], sv, mask=m)
        plsc.store_scatter(ring_d, [pos], dv - lo, mask=m)
        plsc.store_scatter(ring_w, [pos], wv, mask=m)
        cnt = cnt + tot
        cond = (cnt - flushed) >= FLUSH
        pl.when(cond)(lambda: flush(flushed))
        flushed = jnp.where(cond, flushed + FLUSH, flushed)
        return cnt, flushed

    def stage(base, n):
        boff = s * ECHUNK + base
        pltpu.sync_copy(src_h.at[pl.ds(pl.multiple_of(boff, 8), n)], sbuf.at[pl.ds(0, n)])
        pltpu.sync_copy(dst_h.at[pl.ds(pl.multiple_of(boff, 8), n)], dbuf.at[pl.ds(0, n)])
        pltpu.sync_copy(w_h.at[pl.ds(pl.multiple_of(boff, 8), n)], wbuf.at[pl.ds(0, n)])

    def blk_body(b, carry):
        stage(b * BLK, BLK)
        return lax.fori_loop(0, BLK // 16, vbody, carry)

    cnt, flushed = lax.fori_loop(0, NFULL, blk_body,
                                 (jnp.int32(0), jnp.int32(0)))
    stage(NFULL * BLK, TAIL)
    cnt, flushed = lax.fori_loop(0, TAIL // 16, vbody, (cnt, flushed))

    npad = (SUPER - (cnt & (SUPER - 1))) & (SUPER - 1)
    zi = jnp.zeros((16,), jnp.int32)
    zf = jnp.zeros((16,), jnp.float32)
    dumv = jnp.full((16,), DUMMY, jnp.int32)

    def pad_body(i, _):
        idx = i * 16 + lane
        m = idx < npad
        pos = (cnt + idx) & (RING - 1)
        plsc.store_scatter(ring_s, [pos], zi, mask=m)
        plsc.store_scatter(ring_d, [pos], dumv, mask=m)
        plsc.store_scatter(ring_w, [pos], zf, mask=m)
        return 0
    lax.fori_loop(0, SUPER // 16, pad_body, 0)

    total = cnt + npad
    for _ in range(2):
        cond = flushed < total
        pl.when(cond)(functools.partial(flush, flushed))
        flushed = jnp.where(cond, flushed + FLUSH, flushed)

    nsuper = total >> 11
    cntbuf[pl.ds(0, 16)] = jnp.full((16,), nsuper, jnp.int32)
    pltpu.sync_copy(cntbuf, nsup_h.at[pl.ds(pl.multiple_of((c * NSUB + s) * 16, 8), 16)])


_partition = pl.kernel(
    _partition_body,
    out_type=[
        jax.ShapeDtypeStruct((NTILES * CAP,), jnp.int32),
        jax.ShapeDtypeStruct((NTILES * CAP,), jnp.int32),
        jax.ShapeDtypeStruct((NTILES * CAP,), jnp.float32),
        jax.ShapeDtypeStruct((NTILES * 16,), jnp.int32),
    ],
    mesh=plsc.VectorSubcoreMesh(core_axis_name="c", subcore_axis_name="s"),
    compiler_params=pltpu.CompilerParams(needs_layout_passes=False, use_tc_tiling_on_sc=False),
    scratch_types=[
        pltpu.VMEM((BLK,), jnp.int32),
        pltpu.VMEM((BLK,), jnp.int32),
        pltpu.VMEM((BLK,), jnp.float32),
        pltpu.VMEM((RING,), jnp.int32),
        pltpu.VMEM((RING,), jnp.int32),
        pltpu.VMEM((RING,), jnp.float32),
        pltpu.VMEM((16,), jnp.int32),
    ],
)


def _propagate_body(h_h, esrc_h, edst_h, ew_h, nsup_h, zeros_h, xout_h,
                    acc, ls, ld, lw, rows, nbuf,
                    sg0, sg1, sg2, sg3, ss0, ss1, ss2, ss3):
    c = lax.axis_index("c")
    s = lax.axis_index("s")
    lane = lax.iota(jnp.int32, 16)
    semG = (sg0, sg1, sg2, sg3)
    semS = (ss0, ss1, ss2, ss3)

    pltpu.sync_copy(nsup_h.at[pl.ds(pl.multiple_of((c * NSUB + s) * 16, 8), 16)], nbuf)
    nslots = jnp.max(nbuf[...]) * (SUPER // CH)

    pl.when(s < NSUB - 1)(
        lambda: pltpu.sync_copy(zeros_h, acc.at[pl.ds(pl.multiple_of(s * RPT, 8), RPT)]))
    pl.when(s == NSUB - 1)(
        lambda: pltpu.sync_copy(zeros_h.at[pl.ds(0, RPT_LAST)],
                                acc.at[pl.ds((NSUB - 1) * RPT, RPT_LAST)]))
    pl.when(s == 0)(
        lambda: pltpu.sync_copy(zeros_h.at[pl.ds(0, 8)],
                                acc.at[pl.ds(HALF, 8)]))
    plsc.subcore_barrier()

    def load_group(off, gi):
        pltpu.sync_copy(esrc_h.at[c, s, pl.ds(off, 8)], ls.at[gi])
        pltpu.sync_copy(edst_h.at[c, s, pl.ds(off, 8)], ld.at[gi])
        pltpu.sync_copy(ew_h.at[c, s, pl.ds(off, 8)], lw.at[gi])

    def issue_gather(gi, k8, q):
        pltpu.async_copy(h_h.at[ls.at[gi, k8]], rows.at[q], semG[q])

    pl.when(0 < nslots)(functools.partial(load_group, 0, 0))
    for k in range(2):
        pl.when(k < nslots)(functools.partial(issue_gather, 0, k, k))

    def sub(i, k):
        g = i * 16 + k
        q = k & 3
        gi, k8 = (k >> 3) & 1, k & 7
        pltpu.make_async_copy(h_h.at[ls.at[gi, k8]], rows.at[q], semG[q]).wait()

        def grp_body(t, _):
            wv16 = lw[gi, k8, pl.ds(t * 16, 16)]
            for u in range(16):
                e = t * 16 + u
                w = wv16[u]
                rows[q, e, pl.ds(0, 16)] = rows[q, e, pl.ds(0, 16)] * w
                rows[q, e, pl.ds(16, 16)] = rows[q, e, pl.ds(16, 16)] * w
            return 0
        lax.fori_loop(0, CH // 16, grp_body, 0)

        pltpu.async_copy(rows.at[q], acc.at[ld.at[gi, k8]], semS[q], add=True)

        if k == 5:
            pl.when(g + 3 < nslots)(
                functools.partial(load_group, i * 16 + 8, 1))
        if k == 13:
            pl.when(g + 3 < nslots)(
                functools.partial(load_group, i * 16 + 16, 0))

        kp = (k - 2) & 15
        gp, kp8 = (kp >> 3) & 1, kp & 7
        r = (k + 2) & 3
        pl.when((g + 2 < nslots) & (g >= 2))(
            lambda: pltpu.make_async_copy(
                rows.at[r], acc.at[ld.at[gp, kp8]], semS[r]).wait())
        kn = (k + 2) & 15
        gn, kn8 = (kn >> 3) & 1, kn & 7
        pl.when(g + 2 < nslots)(functools.partial(issue_gather, gn, kn8, r))
        return 0

    def body16(i, _):
        for k in range(16):
            sub(i, k)
        return 0

    lax.fori_loop(0, nslots // 16, body16, 0)
    for j in range(DEPTH):
        pl.when(j < nslots)(
            functools.partial(
                lambda jj: pltpu.make_async_copy(
                    rows.at[jj], acc.at[ld.at[1, 4 + jj]], semS[jj]).wait(), j))
    plsc.subcore_barrier()
    pl.when(s < NSUB - 1)(
        lambda: pltpu.sync_copy(acc.at[pl.ds(pl.multiple_of(s * RPT, 8), RPT)],
                                xout_h.at[pl.ds(pl.multiple_of(c * HALF + s * RPT, 8), RPT)]))
    pl.when(s == NSUB - 1)(
        lambda: pltpu.sync_copy(
            acc.at[pl.ds((NSUB - 1) * RPT, RPT_LAST)],
            xout_h.at[pl.ds(pl.multiple_of(c * HALF + (NSUB - 1) * RPT, 8), RPT_LAST)]))


_propagate = pl.kernel(
    _propagate_body,
    out_type=jax.ShapeDtypeStruct((N_NODES, D), jnp.float32),
    mesh=plsc.VectorSubcoreMesh(core_axis_name="c", subcore_axis_name="s"),
    compiler_params=pltpu.CompilerParams(needs_layout_passes=False, use_tc_tiling_on_sc=False),
    scratch_types=[
        pltpu.VMEM_SHARED((ACC_ROWS, D), jnp.float32),
        pltpu.VMEM((2, 8, CH), jnp.int32),
        pltpu.VMEM((2, 8, CH), jnp.int32),
        pltpu.VMEM((2, 8, CH), jnp.float32),
        pltpu.VMEM((DEPTH, CH, D), jnp.float32),
        pltpu.VMEM((16,), jnp.int32),
    ] + [pltpu.SemaphoreType.DMA] * (2 * DEPTH),
)


def _linear_kernel(x_ref, w_ref, b_ref, o_ref, *, relu):
    x = x_ref[...]
    if relu:
        x = jnp.maximum(x, 0.0)
    o_ref[...] = lax.dot_general(
        x, w_ref[...], (((1,), (1,)), ((), ())),
        preferred_element_type=jnp.float32) + b_ref[...]


def _linear(x, w, b2, relu):
    rb = 4000
    return pl.pallas_call(
        functools.partial(_linear_kernel, relu=relu),
        out_shape=jax.ShapeDtypeStruct((N_NODES, D), jnp.float32),
        grid=(N_NODES // rb,),
        in_specs=[
            pl.BlockSpec((rb, D), lambda i: (i, 0)),
            pl.BlockSpec((D, D), lambda i: (0, 0)),
            pl.BlockSpec((1, D), lambda i: (0, 0)),
        ],
        out_specs=pl.BlockSpec((rb, D), lambda i: (i, 0)),
    )(x, w, b2)


def _relu_kernel(x_ref, o_ref):
    o_ref[...] = jnp.maximum(x_ref[...], 0.0)


def _relu(x):
    rb = 4000
    return pl.pallas_call(
        _relu_kernel,
        out_shape=jax.ShapeDtypeStruct((N_NODES, D), jnp.float32),
        grid=(N_NODES // rb,),
        in_specs=[pl.BlockSpec((rb, D), lambda i: (i, 0))],
        out_specs=pl.BlockSpec((rb, D), lambda i: (i, 0)),
    )(x)


def kernel(edge_index, edge_weight, row_embed, col_embed,
           W0, b0, W1, b1, W2, b2):
    src = edge_index[0].astype(jnp.int32)
    dst = edge_index[1].astype(jnp.int32)
    esrc, edst, ew, nsup = _partition(src, dst, edge_weight)
    esrc5 = esrc.reshape(NCORES, NSUB, NSLOT, CH)
    edst5 = edst.reshape(NCORES, NSUB, NSLOT, CH)
    ew5 = ew.reshape(NCORES, NSUB, NSLOT, CH)
    zeros = jnp.zeros((RPT, D), jnp.float32)
    x = jnp.concatenate([row_embed, col_embed], axis=0)
    for i, (W, b) in enumerate(((W0, b0), (W1, b1), (W2, b2))):
        h = _linear(x, W, b.reshape(1, D), relu=(i > 0))
        x = _propagate(h, esrc5, edst5, ew5, nsup, zeros)
    x = _relu(x)
    return (x[:N_ROWS], x[N_ROWS:])
